# Initial kernel scaffold; baseline (speedup 1.0000x reference)
import functools

import jax
import jax.numpy as jnp
from jax import lax
from jax.experimental import pallas as pl
from jax.experimental.pallas import tpu as pltpu
from jax.experimental.pallas import tpu_sc as plsc

N_NODES = 10000
N_EDGES = 320000
FEAT = 128

NC = 2
NS = 16
NW = NC * NS
BATCH = 128
ROWS_PER_TILE = (N_EDGES + NW * BATCH - 1) // (NW * BATCH)
E_PAD = NW * ROWS_PER_TILE * BATCH
ACC_ROWS = 10240
ZERO_CHUNK = 128
OUT_PER_TILE = N_NODES // NS


def _sc_body(x_hbm, src_hbm, dst_hbm, out_hbm, sidx, didx, rows, accum, sem):
    c = lax.axis_index("c")
    s = lax.axis_index("s")
    wid = s * NC + c

    pltpu.sync_copy(src_hbm.at[wid], sidx)
    pltpu.sync_copy(dst_hbm.at[wid], didx)

    def _zero_row(r, _):
        for cc in range(FEAT // 16):
            rows[r, pl.ds(cc * 16, 16)] = jnp.zeros((16,), jnp.float32)
        return 0

    lax.fori_loop(0, ZERO_CHUNK, _zero_row, 0)
    for k in range(5):
        pltpu.sync_copy(rows, accum.at[pl.ds(s * 640 + k * ZERO_CHUNK, ZERO_CHUNK)])
    plsc.subcore_barrier()

    def _step(j, _):
        pltpu.async_copy(x_hbm.at[sidx.at[j]], rows, sem).wait()
        pltpu.sync_copy(rows, accum.at[didx.at[j]], add=True)
        return 0

    lax.fori_loop(0, ROWS_PER_TILE, _step, 0)
    plsc.subcore_barrier()

    pltpu.sync_copy(
        accum.at[pl.ds(s * OUT_PER_TILE, OUT_PER_TILE)],
        out_hbm.at[c, pl.ds(s * OUT_PER_TILE, OUT_PER_TILE)],
    )


@jax.jit
def _sc_spmm(x, src_r, dst_r):
    mesh = plsc.VectorSubcoreMesh(core_axis_name="c", subcore_axis_name="s")
    return pl.kernel(
        _sc_body,
        out_type=jax.ShapeDtypeStruct((NC, N_NODES, FEAT), jnp.float32),
        mesh=mesh,
        scratch_types=[
            pltpu.VMEM((ROWS_PER_TILE, BATCH), jnp.int32),
            pltpu.VMEM((ROWS_PER_TILE, BATCH), jnp.int32),
            pltpu.VMEM((BATCH, FEAT), jnp.float32),
            pltpu.VMEM_SHARED((ACC_ROWS, FEAT), jnp.float32),
            pltpu.SemaphoreType.DMA,
        ],
    )(x, src_r, dst_r)


def _tc_body(p0_ref, p1_ref, w_ref, b_ref, o_ref):
    h = p0_ref[...] + p1_ref[...]
    o_ref[...] = (
        jnp.dot(h, w_ref[...], preferred_element_type=jnp.float32) + b_ref[...]
    )


@jax.jit
def _tc_combine(p0, p1, W, b):
    blk = 1000
    grid = (N_NODES // blk,)
    return pl.pallas_call(
        _tc_body,
        grid=grid,
        in_specs=[
            pl.BlockSpec((blk, FEAT), lambda i: (i, 0)),
            pl.BlockSpec((blk, FEAT), lambda i: (i, 0)),
            pl.BlockSpec((FEAT, FEAT), lambda i: (0, 0)),
            pl.BlockSpec((1, FEAT), lambda i: (0, 0)),
        ],
        out_specs=pl.BlockSpec((blk, FEAT), lambda i: (i, 0)),
        out_shape=jax.ShapeDtypeStruct((N_NODES, FEAT), jnp.float32),
    )(p0, p1, W, b)


def kernel(x, edge_index, W, b):
    src = edge_index[0].astype(jnp.int32)
    dst = edge_index[1].astype(jnp.int32)
    pad = E_PAD - N_EDGES
    src_r = jnp.concatenate([src, jnp.zeros((pad,), jnp.int32)]).reshape(
        NW, ROWS_PER_TILE, BATCH
    )
    dst_r = jnp.concatenate(
        [dst, jnp.full((pad,), N_NODES, jnp.int32)]
    ).reshape(NW, ROWS_PER_TILE, BATCH)
    partial = _sc_spmm(x, src_r, dst_r)
    return _tc_combine(partial[0], partial[1], W, b)


# speedup vs baseline: 4.7384x; 4.7384x over previous
import functools

import jax
import jax.numpy as jnp
from jax import lax
from jax.experimental import pallas as pl
from jax.experimental.pallas import tpu as pltpu
from jax.experimental.pallas import tpu_sc as plsc

N_NODES = 10000
N_EDGES = 320000
FEAT = 128

NC = 2
NS = 16
NW = NC * NS
BATCH = 128
ROWS_PER_TILE = (N_EDGES + NW * BATCH - 1) // (NW * BATCH)
E_PAD = NW * ROWS_PER_TILE * BATCH
ACC_ROWS = 10240
ZERO_CHUNK = 128
OUT_PER_TILE = ACC_ROWS // NS


def _sc_body(x_hbm, src_hbm, dst_hbm, out_hbm, sidx, didx, rows, accum, sem):
    c = lax.axis_index("c")
    s = lax.axis_index("s")
    wid = s * NC + c

    pltpu.sync_copy(src_hbm.at[wid], sidx)
    pltpu.sync_copy(dst_hbm.at[wid], didx)

    def _zero_row(r, _):
        for cc in range(FEAT // 16):
            rows[r, pl.ds(cc * 16, 16)] = jnp.zeros((16,), jnp.float32)
        return 0

    lax.fori_loop(0, ZERO_CHUNK, _zero_row, 0)
    for k in range(5):
        pltpu.sync_copy(rows, accum.at[pl.ds(s * 640 + k * ZERO_CHUNK, ZERO_CHUNK)])
    plsc.subcore_barrier()

    def _step(j, _):
        pltpu.async_copy(x_hbm.at[sidx.at[j]], rows, sem).wait()
        pltpu.sync_copy(rows, accum.at[didx.at[j]], add=True)
        return 0

    lax.fori_loop(0, ROWS_PER_TILE, _step, 0)
    plsc.subcore_barrier()

    pltpu.sync_copy(
        accum.at[pl.ds(s * OUT_PER_TILE, OUT_PER_TILE)],
        out_hbm.at[c, pl.ds(s * OUT_PER_TILE, OUT_PER_TILE)],
    )


@jax.jit
def _sc_spmm(x, src_r, dst_r):
    mesh = plsc.VectorSubcoreMesh(core_axis_name="c", subcore_axis_name="s")
    return pl.kernel(
        _sc_body,
        out_type=jax.ShapeDtypeStruct((NC, ACC_ROWS, FEAT), jnp.float32),
        mesh=mesh,
        scratch_types=[
            pltpu.VMEM((ROWS_PER_TILE, BATCH), jnp.int32),
            pltpu.VMEM((ROWS_PER_TILE, BATCH), jnp.int32),
            pltpu.VMEM((BATCH, FEAT), jnp.float32),
            pltpu.VMEM_SHARED((ACC_ROWS, FEAT), jnp.float32),
            pltpu.SemaphoreType.DMA,
        ],
    )(x, src_r, dst_r)


def _tc_body(p0_ref, p1_ref, w_ref, b_ref, o_ref):
    h = p0_ref[0] + p1_ref[0]
    o_ref[...] = (
        jnp.dot(h, w_ref[...], preferred_element_type=jnp.float32) + b_ref[...]
    )


@jax.jit
def _tc_combine(partial, W, b):
    blk = 1000
    grid = (N_NODES // blk,)
    return pl.pallas_call(
        _tc_body,
        grid=grid,
        in_specs=[
            pl.BlockSpec((1, blk, FEAT), lambda i: (0, i, 0)),
            pl.BlockSpec((1, blk, FEAT), lambda i: (1, i, 0)),
            pl.BlockSpec((FEAT, FEAT), lambda i: (0, 0)),
            pl.BlockSpec((1, FEAT), lambda i: (0, 0)),
        ],
        out_specs=pl.BlockSpec((blk, FEAT), lambda i: (i, 0)),
        out_shape=jax.ShapeDtypeStruct((N_NODES, FEAT), jnp.float32),
    )(partial, partial, W, b)


def kernel(x, edge_index, W, b):
    src = edge_index[0].astype(jnp.int32)
    dst = edge_index[1].astype(jnp.int32)
    pad = E_PAD - N_EDGES
    src_r = jnp.concatenate([src, jnp.zeros((pad,), jnp.int32)]).reshape(
        NW, ROWS_PER_TILE, BATCH
    )
    dst_r = jnp.concatenate(
        [dst, jnp.full((pad,), N_NODES, jnp.int32)]
    ).reshape(NW, ROWS_PER_TILE, BATCH)
    partial = _sc_spmm(x, src_r, dst_r)
    return _tc_combine(partial, W, b)
